# baseline (speedup 1.0000x reference)
import jax
import jax.numpy as jnp
from jax.experimental import pallas as pl
from jax.experimental.pallas import tpu as pltpu

_NUM = 10
_CBLK = 2


def _thermo_block(x_ref, o_ref):
    x = x_ref[...]
    k = jax.lax.broadcasted_iota(jnp.int32, (1, _NUM, 1, 1), 1)
    thr = jnp.where(k == 0, -1.0, k.astype(jnp.float32) / _NUM)
    o_ref[...] = (x[:, None, :, :] > thr).astype(x.dtype)


_HBLK = 256


def kernel(x):
    B, C, H, W = x.shape
    xf = x.reshape(B * C, H, W)
    out = pl.pallas_call(
        _thermo_block,
        grid=(B * C // _CBLK, H // _HBLK),
        in_specs=[pl.BlockSpec((_CBLK, _HBLK, W), lambda i, j: (i, j, 0))],
        out_specs=pl.BlockSpec((_CBLK, _NUM, _HBLK, W), lambda i, j: (i, 0, j, 0)),
        out_shape=jax.ShapeDtypeStruct((B * C, _NUM, H, W), x.dtype),
        compiler_params=pltpu.CompilerParams(
            dimension_semantics=("parallel", "parallel"),
            vmem_limit_bytes=56 * 1024 * 1024,
        ),
    )(xf)
    return out.reshape(B, C * _NUM, H, W)


# speedup vs baseline: 1.0153x; 1.0153x over previous
import jax
import jax.numpy as jnp
from jax.experimental import pallas as pl
from jax.experimental.pallas import tpu as pltpu

_NUM = 10
_CBLK = 2


def _thermo_block(x_ref, o_ref):
    x = x_ref[...]
    k = jax.lax.broadcasted_iota(jnp.int32, (1, _NUM, 1, 1), 1)
    thr = jnp.where(k == 0, -1.0, k.astype(jnp.float32) / _NUM)
    o_ref[...] = (x[:, None, :, :] > thr).astype(x.dtype)


def kernel(x):
    B, C, H, W = x.shape
    xf = x.reshape(B * C, H, W)
    out = pl.pallas_call(
        _thermo_block,
        grid=(B * C // _CBLK,),
        in_specs=[pl.BlockSpec((_CBLK, H, W), lambda i: (i, 0, 0))],
        out_specs=pl.BlockSpec((_CBLK, _NUM, H, W), lambda i: (i, 0, 0, 0)),
        out_shape=jax.ShapeDtypeStruct((B * C, _NUM, H, W), x.dtype),
        compiler_params=pltpu.CompilerParams(
            dimension_semantics=("parallel",),
            vmem_limit_bytes=56 * 1024 * 1024,
        ),
    )(xf)
    return out.reshape(B, C * _NUM, H, W)
